# Initial kernel scaffold; baseline (speedup 1.0000x reference)
import jax
import jax.numpy as jnp
from jax.experimental import pallas as pl


def kernel(x, edge_index, edge_type, graph_ids, W1, b1, W2, b2):
    raise NotImplementedError("write your pallas kernel here")


# speedup vs baseline: 3.4346x; 3.4346x over previous
import functools

import jax
import jax.numpy as jnp
from jax import lax
from jax.experimental import pallas as pl
from jax.experimental.pallas import tpu as pltpu
from jax.experimental.pallas import tpu_sc as plsc

N = 10000
E = 320000
D = 128
H = 128
Hh = H // 2
R = 3
G = 64

NSC = 2
NT = 16
Nh = N // NSC

SEG = 6144
GARB = 5000
ACC_ROWS = R * SEG
TSLICE = ACC_ROWS // NT
ZCH = 96

DR = 30720
DEGSZ = 2 * DR
DGARB = R * N
DSLICE = DEGSZ // NT

EPAD = 327680
CH = 128

_f32 = jnp.float32
_i32 = jnp.int32
_HIGH = jax.lax.Precision.HIGHEST


@functools.cache
def _vmesh():
    return plsc.VectorSubcoreMesh(core_axis_name="c", subcore_axis_name="s",
                                  num_cores=NSC, num_subcores=NT)


def _k0_body(src, dst, typ, fsrc_g, fsrc_d, fdst_d, lidx0, lidx1):
    s = src[...]
    d = dst[...]
    t = typ[...]
    real = d < N
    fsrc_g[...] = t * N + s
    fsrc_d[...] = jnp.where(real, t * N + s, DGARB)
    fdst_d[...] = jnp.where(real, DR + t * N + d, DGARB)
    lidx0[...] = jnp.where(d < Nh, t * SEG + d, GARB)
    lidx1[...] = jnp.where(real & (d >= Nh), t * SEG + (d - Nh), GARB)


def _k0(src, dst, typ, interpret=False):
    shp = jax.ShapeDtypeStruct(src.shape, _i32)
    return pl.pallas_call(
        _k0_body,
        out_shape=[shp] * 5,
        interpret=interpret,
    )(src, dst, typ)


def _k1_body(fsrc_hbm, fdst_hbm, out_hbm, idx_a, idx_b, ones_v, zero_v,
             degacc):
    c = lax.axis_index("c")
    s = lax.axis_index("s")

    @pl.loop(0, CH, step=16)
    def _(i):
        ones_v[pl.ds(i, 16)] = jnp.ones((16,), _f32)
        zero_v[pl.ds(i, 16)] = jnp.zeros((16,), _f32)

    @pl.loop(0, DSLICE, step=CH)
    def _(i):
        pltpu.sync_copy(zero_v, degacc.at[pl.ds(s * DSLICE + i, CH)])

    plsc.subcore_barrier()

    base = (c * NT + s) * (EPAD // (NSC * NT))

    @pl.loop(0, EPAD // (NSC * NT), step=CH)
    def _(i):
        pltpu.sync_copy(fsrc_hbm.at[pl.ds(base + i, CH)], idx_a)
        pltpu.sync_copy(fdst_hbm.at[pl.ds(base + i, CH)], idx_b)
        pltpu.sync_copy(ones_v, degacc.at[idx_a], add=True)
        pltpu.sync_copy(ones_v, degacc.at[idx_b], add=True)

    plsc.subcore_barrier()
    pltpu.sync_copy(degacc.at[pl.ds(s * DSLICE, DSLICE)],
                    out_hbm.at[c, pl.ds(s * DSLICE, DSLICE)])


@functools.cache
def _k1_built():
    return pl.kernel(
        _k1_body,
        out_type=jax.ShapeDtypeStruct((NSC, DEGSZ), _f32),
        mesh=_vmesh(),
        scratch_types=[
            pltpu.VMEM((CH,), _i32),
            pltpu.VMEM((CH,), _i32),
            pltpu.VMEM((CH,), _f32),
            pltpu.VMEM((CH,), _f32),
            pltpu.VMEM_SHARED((DEGSZ,), _f32),
        ],
    )


def _k1(fsrc_d, fdst_d):
    return _k1_built()(fsrc_d, fdst_d)


def _ksc_agg_body(ftab_hbm, fsrc_hbm, lidx_hbm, out_hbm, gi, si, rows, zbuf,
                  acc):
    c = lax.axis_index("c")
    s = lax.axis_index("s")

    @pl.loop(0, ZCH)
    def _(r):
        @pl.loop(0, Hh, step=16)
        def _(l):
            zbuf[r, pl.ds(l, 16)] = jnp.zeros((16,), _f32)

    @pl.loop(0, TSLICE, step=ZCH)
    def _(i):
        pltpu.sync_copy(zbuf, acc.at[pl.ds(s * TSLICE + i, ZCH)])

    plsc.subcore_barrier()

    base = s * (EPAD // NT)

    @pl.loop(0, EPAD // NT, step=CH)
    def _(i):
        pltpu.sync_copy(fsrc_hbm.at[pl.ds(base + i, CH)], gi)
        pltpu.sync_copy(lidx_hbm.at[c, pl.ds(base + i, CH)], si)
        pltpu.sync_copy(ftab_hbm.at[gi], rows)
        pltpu.sync_copy(rows, acc.at[si], add=True)

    plsc.subcore_barrier()
    pltpu.sync_copy(acc.at[pl.ds(s * TSLICE, TSLICE)],
                    out_hbm.at[c, pl.ds(s * TSLICE, TSLICE)])


@functools.cache
def _ksc_agg_built():
    return pl.kernel(
        _ksc_agg_body,
        out_type=jax.ShapeDtypeStruct((NSC, ACC_ROWS, Hh), _f32),
        mesh=_vmesh(),
        compiler_params=pltpu.CompilerParams(use_tc_tiling_on_sc=False),
        scratch_types=[
            pltpu.VMEM((CH,), _i32),
            pltpu.VMEM((CH,), _i32),
            pltpu.VMEM((CH, Hh), _f32),
            pltpu.VMEM((ZCH, Hh), _f32),
            pltpu.VMEM_SHARED((ACC_ROWS, Hh), _f32),
        ],
    )


def _ksc_agg(ftab, fsrc, lidx):
    return _ksc_agg_built()(ftab, fsrc, lidx)


def _k2_body(x, da, db, outA, outB):
    deg = da[...] + db[...]
    norm = lax.rsqrt(jnp.maximum(deg, 1.0))
    f = x[...] * norm
    outA[...] = f[:, :Hh]
    outB[...] = f[:, Hh:]


def _k2(x, degA, degB, interpret=False):
    nb = R * N // 1000
    return pl.pallas_call(
        _k2_body,
        grid=(nb,),
        in_specs=[
            pl.BlockSpec((1000, D), lambda i: (i % (N // 1000), 0)),
            pl.BlockSpec((1000, 1), lambda i: (i, 0)),
            pl.BlockSpec((1000, 1), lambda i: (i, 0)),
        ],
        out_specs=[pl.BlockSpec((1000, Hh), lambda i: (i, 0))] * 2,
        out_shape=[jax.ShapeDtypeStruct((R * N, Hh), _f32)] * 2,
        interpret=interpret,
    )(x, degA, degB)


def _agg_spec():
    return pl.BlockSpec((1, R, 1000, Hh),
                        lambda j: (j // (Nh // 1000), 0, j % (Nh // 1000), 0))


def _k4_body(aggA, aggB, dia, dib, doa, dob, W, b, outA, outB):
    agg = jnp.concatenate([aggA[0], aggB[0]], axis=-1)
    ndst = lax.rsqrt(jnp.maximum(dia[...] + dib[...], 1.0))
    h = jnp.zeros((1000, H), _f32)
    for r in range(R):
        h = h + jnp.dot(agg[r] * ndst[r], W[r], precision=_HIGH,
                        preferred_element_type=_f32)
    h = h + jnp.sum(b[...], axis=0)
    h = jnp.maximum(h, 0.0)
    nsrc = lax.rsqrt(jnp.maximum(doa[...] + dob[...], 1.0))
    f = h[None, :, :] * nsrc
    outA[...] = f[..., :Hh]
    outB[...] = f[..., Hh:]


def _k4(aggA, aggB, diA, diB, doA, doB, W, b, interpret=False):
    nb = N // 1000
    return pl.pallas_call(
        _k4_body,
        grid=(nb,),
        in_specs=[
            _agg_spec(),
            _agg_spec(),
            pl.BlockSpec((R, 1000, 1), lambda j: (0, j, 0)),
            pl.BlockSpec((R, 1000, 1), lambda j: (0, j, 0)),
            pl.BlockSpec((R, 1000, 1), lambda j: (0, j, 0)),
            pl.BlockSpec((R, 1000, 1), lambda j: (0, j, 0)),
            pl.BlockSpec((R, H, H), lambda j: (0, 0, 0)),
            pl.BlockSpec((R, 1, H), lambda j: (0, 0, 0)),
        ],
        out_specs=[pl.BlockSpec((R, 1000, Hh), lambda j: (0, j, 0))] * 2,
        out_shape=[jax.ShapeDtypeStruct((R, N, Hh), _f32)] * 2,
        interpret=interpret,
    )(aggA, aggB, diA, diB, doA, doB, W, b)


def _k6_body(aggA, aggB, dia, dib, W, b, gids, out, sums, counts):
    j = pl.program_id(0)
    nb = pl.num_programs(0)
    agg = jnp.concatenate([aggA[0], aggB[0]], axis=-1)
    ndst = lax.rsqrt(jnp.maximum(dia[...] + dib[...], 1.0))
    h = jnp.zeros((1000, H), _f32)
    for r in range(R):
        h = h + jnp.dot(agg[r] * ndst[r], W[r], precision=_HIGH,
                        preferred_element_type=_f32)
    h = h + jnp.sum(b[...], axis=0)
    giota = lax.broadcasted_iota(_i32, (1000, G), 1)
    P = (gids[...] == giota).astype(_f32)
    dn = (((0,), (0,)), ((), ()))
    psum = lax.dot_general(P, h, dn, precision=_HIGH,
                           preferred_element_type=_f32)
    pcnt = lax.dot_general(P, jnp.ones((1000, H), _f32), dn, precision=_HIGH,
                           preferred_element_type=_f32)

    @pl.when(j == 0)
    def _():
        sums[...] = psum
        counts[...] = pcnt

    @pl.when(j > 0)
    def _():
        sums[...] += psum
        counts[...] += pcnt

    @pl.when(j == nb - 1)
    def _():
        out[...] = sums[...] / jnp.maximum(counts[...], 1.0)


def _k6(aggA, aggB, diA, diB, W, b, gids, interpret=False):
    nb = N // 1000
    return pl.pallas_call(
        _k6_body,
        grid=(nb,),
        in_specs=[
            _agg_spec(),
            _agg_spec(),
            pl.BlockSpec((R, 1000, 1), lambda j: (0, j, 0)),
            pl.BlockSpec((R, 1000, 1), lambda j: (0, j, 0)),
            pl.BlockSpec((R, H, H), lambda j: (0, 0, 0)),
            pl.BlockSpec((R, 1, H), lambda j: (0, 0, 0)),
            pl.BlockSpec((1000, 1), lambda j: (j, 0)),
        ],
        out_specs=pl.BlockSpec((G, H), lambda j: (0, 0)),
        out_shape=jax.ShapeDtypeStruct((G, H), _f32),
        scratch_shapes=[pltpu.VMEM((G, H), _f32), pltpu.VMEM((G, H), _f32)],
        interpret=interpret,
    )(aggA, aggB, diA, diB, W, b, gids)


def kernel(x, edge_index, edge_type, graph_ids, W1, b1, W2, b2):
    src = edge_index[0].astype(_i32)
    dst = edge_index[1].astype(_i32)
    typ = edge_type.astype(_i32)

    pad = EPAD - E
    src_p = jnp.pad(src, (0, pad))
    dst_p = jnp.pad(dst, (0, pad), constant_values=10**8)
    typ_p = jnp.pad(typ, (0, pad))

    idx = _k0(src_p.reshape(-1, 128), dst_p.reshape(-1, 128),
              typ_p.reshape(-1, 128))
    fsrc_g, fsrc_d, fdst_d, lidx0, lidx1 = [a.reshape(-1) for a in idx]
    lidx = jnp.stack([lidx0, lidx1], axis=0)

    degp = _k1(fsrc_d, fdst_d)
    doA = degp[0, :R * N].reshape(R * N, 1)
    doB = degp[1, :R * N].reshape(R * N, 1)
    diA = degp[0, DR:DR + R * N].reshape(R * N, 1)
    diB = degp[1, DR:DR + R * N].reshape(R * N, 1)
    diA3 = diA.reshape(R, N, 1)
    diB3 = diB.reshape(R, N, 1)
    doA3 = doA.reshape(R, N, 1)
    doB3 = doB.reshape(R, N, 1)

    F1a, F1b = _k2(x, doA, doB)
    o1a = _ksc_agg(F1a, fsrc_g, lidx).reshape(NSC, R, SEG, Hh)
    o1b = _ksc_agg(F1b, fsrc_g, lidx).reshape(NSC, R, SEG, Hh)

    F2a, F2b = _k4(o1a, o1b, diA3, diB3, doA3, doB3, W1, b1.reshape(R, 1, H))
    o2a = _ksc_agg(F2a.reshape(R * N, Hh), fsrc_g, lidx).reshape(
        NSC, R, SEG, Hh)
    o2b = _ksc_agg(F2b.reshape(R * N, Hh), fsrc_g, lidx).reshape(
        NSC, R, SEG, Hh)

    hg = _k6(o2a, o2b, diA3, diB3, W2, b2.reshape(R, 1, H),
             graph_ids.astype(_i32).reshape(N, 1))
    return hg


# speedup vs baseline: 4.0871x; 1.1900x over previous
import functools

import jax
import jax.numpy as jnp
from jax import lax
from jax.experimental import pallas as pl
from jax.experimental.pallas import tpu as pltpu
from jax.experimental.pallas import tpu_sc as plsc

N = 10000
E = 320000
D = 128
H = 128
Hh = H // 2
R = 3
G = 64

NSC = 2
NT = 16
Nh = N // NSC

SEG = 5120
GARB = 5000
ACC_ROWS = R * SEG
TSLICE = ACC_ROWS // NT
ZCH = 96

DR = 30720
DEGSZ = 2 * DR
DGARB = R * N
DSLICE = DEGSZ // NT

EPAD = 327680
CH = 128
NCHT = EPAD // (NT * CH)

_f32 = jnp.float32
_i32 = jnp.int32
_HIGH = jax.lax.Precision.HIGHEST


@functools.cache
def _vmesh():
    return plsc.VectorSubcoreMesh(core_axis_name="c", subcore_axis_name="s",
                                  num_cores=NSC, num_subcores=NT)


def _k0_body(src, dst, typ, fsrc_g, fsrc_d, fdst_d, lidx0, lidx1):
    s = src[...]
    d = dst[...]
    t = typ[...]
    real = d < N
    fsrc_g[...] = t * N + s
    fsrc_d[...] = jnp.where(real, t * N + s, DGARB)
    fdst_d[...] = jnp.where(real, DR + t * N + d, DGARB)
    lidx0[...] = jnp.where(d < Nh, t * SEG + d, GARB)
    lidx1[...] = jnp.where(real & (d >= Nh), t * SEG + (d - Nh), GARB)


def _k0(src, dst, typ, interpret=False):
    shp = jax.ShapeDtypeStruct(src.shape, _i32)
    return pl.pallas_call(
        _k0_body,
        out_shape=[shp] * 5,
        interpret=interpret,
    )(src, dst, typ)


def _k1_body(fsrc_hbm, fdst_hbm, out_hbm, idx_a, idx_b, ones_v, zero_v,
             degacc):
    c = lax.axis_index("c")
    s = lax.axis_index("s")

    @pl.loop(0, CH, step=16)
    def _(i):
        ones_v[pl.ds(i, 16)] = jnp.ones((16,), _f32)
        zero_v[pl.ds(i, 16)] = jnp.zeros((16,), _f32)

    @pl.loop(0, DSLICE, step=CH)
    def _(i):
        pltpu.sync_copy(zero_v, degacc.at[pl.ds(s * DSLICE + i, CH)])

    plsc.subcore_barrier()

    base = (c * NT + s) * (EPAD // (NSC * NT))

    @pl.loop(0, EPAD // (NSC * NT), step=CH)
    def _(i):
        pltpu.sync_copy(fsrc_hbm.at[pl.ds(base + i, CH)], idx_a)
        pltpu.sync_copy(fdst_hbm.at[pl.ds(base + i, CH)], idx_b)
        pltpu.sync_copy(ones_v, degacc.at[idx_a], add=True)
        pltpu.sync_copy(ones_v, degacc.at[idx_b], add=True)

    plsc.subcore_barrier()
    pltpu.sync_copy(degacc.at[pl.ds(s * DSLICE, DSLICE)],
                    out_hbm.at[c, pl.ds(s * DSLICE, DSLICE)])


@functools.cache
def _k1_built():
    return pl.kernel(
        _k1_body,
        out_type=jax.ShapeDtypeStruct((NSC, DEGSZ), _f32),
        mesh=_vmesh(),
        scratch_types=[
            pltpu.VMEM((CH,), _i32),
            pltpu.VMEM((CH,), _i32),
            pltpu.VMEM((CH,), _f32),
            pltpu.VMEM((CH,), _f32),
            pltpu.VMEM_SHARED((DEGSZ,), _f32),
        ],
    )


def _k1(fsrc_d, fdst_d):
    return _k1_built()(fsrc_d, fdst_d)


def _ksc_agg_body(ftab_hbm, fsrc_hbm, lidx_hbm, out_hbm, gi, si, rows_a,
                  rows_b, zbuf, acc, sem_a, sem_b):
    c = lax.axis_index("c")
    s = lax.axis_index("s")

    pltpu.sync_copy(fsrc_hbm.at[pl.ds(s * NCHT, NCHT)], gi)
    pltpu.sync_copy(lidx_hbm.at[c, pl.ds(s * NCHT, NCHT)], si)

    @pl.loop(0, ZCH)
    def _(r):
        @pl.loop(0, Hh, step=16)
        def _(l):
            zbuf[r, pl.ds(l, 16)] = jnp.zeros((16,), _f32)

    @pl.loop(0, TSLICE, step=ZCH)
    def _(i):
        pltpu.sync_copy(zbuf, acc.at[pl.ds(s * TSLICE + i, ZCH)])

    plsc.subcore_barrier()

    pltpu.async_copy(ftab_hbm.at[gi.at[0]], rows_a, sem_a)

    @pl.loop(0, NCHT - 2, step=2)
    def _(j):
        pltpu.async_copy(ftab_hbm.at[gi.at[j + 1]], rows_b, sem_b)
        pltpu.make_async_copy(ftab_hbm.at[gi.at[j]], rows_a, sem_a).wait()
        pltpu.sync_copy(rows_a, acc.at[si.at[j]], add=True)
        pltpu.async_copy(ftab_hbm.at[gi.at[j + 2]], rows_a, sem_a)
        pltpu.make_async_copy(ftab_hbm.at[gi.at[j + 1]], rows_b, sem_b).wait()
        pltpu.sync_copy(rows_b, acc.at[si.at[j + 1]], add=True)

    pltpu.async_copy(ftab_hbm.at[gi.at[NCHT - 1]], rows_b, sem_b)
    pltpu.make_async_copy(ftab_hbm.at[gi.at[NCHT - 2]], rows_a, sem_a).wait()
    pltpu.sync_copy(rows_a, acc.at[si.at[NCHT - 2]], add=True)
    pltpu.make_async_copy(ftab_hbm.at[gi.at[NCHT - 1]], rows_b, sem_b).wait()
    pltpu.sync_copy(rows_b, acc.at[si.at[NCHT - 1]], add=True)

    plsc.subcore_barrier()
    pltpu.sync_copy(acc.at[pl.ds(s * TSLICE, TSLICE)],
                    out_hbm.at[c, pl.ds(s * TSLICE, TSLICE)])


@functools.cache
def _ksc_agg_built():
    return pl.kernel(
        _ksc_agg_body,
        out_type=jax.ShapeDtypeStruct((NSC, ACC_ROWS, Hh), _f32),
        mesh=_vmesh(),
        compiler_params=pltpu.CompilerParams(use_tc_tiling_on_sc=False),
        scratch_types=[
            pltpu.VMEM((NCHT, CH), _i32),
            pltpu.VMEM((NCHT, CH), _i32),
            pltpu.VMEM((CH, Hh), _f32),
            pltpu.VMEM((CH, Hh), _f32),
            pltpu.VMEM((ZCH, Hh), _f32),
            pltpu.VMEM_SHARED((ACC_ROWS, Hh), _f32),
            pltpu.SemaphoreType.DMA,
            pltpu.SemaphoreType.DMA,
        ],
    )


def _ksc_agg(ftab, fsrc, lidx):
    return _ksc_agg_built()(ftab, fsrc, lidx)


def _k2_body(x, da, db, outA, outB):
    deg = da[...] + db[...]
    norm = lax.rsqrt(jnp.maximum(deg, 1.0))
    f = x[...] * norm
    outA[...] = f[:, :Hh]
    outB[...] = f[:, Hh:]


def _k2(x, degA, degB, interpret=False):
    nb = R * N // 1000
    return pl.pallas_call(
        _k2_body,
        grid=(nb,),
        in_specs=[
            pl.BlockSpec((1000, D), lambda i: (i % (N // 1000), 0)),
            pl.BlockSpec((1000, 1), lambda i: (i, 0)),
            pl.BlockSpec((1000, 1), lambda i: (i, 0)),
        ],
        out_specs=[pl.BlockSpec((1000, Hh), lambda i: (i, 0))] * 2,
        out_shape=[jax.ShapeDtypeStruct((R * N, Hh), _f32)] * 2,
        interpret=interpret,
    )(x, degA, degB)


def _agg_spec():
    return pl.BlockSpec((1, R, 1000, Hh),
                        lambda j: (j // (Nh // 1000), 0, j % (Nh // 1000), 0))


def _k4_body(aggA, aggB, dia, dib, doa, dob, W, b, outA, outB):
    agg = jnp.concatenate([aggA[0], aggB[0]], axis=-1)
    ndst = lax.rsqrt(jnp.maximum(dia[...] + dib[...], 1.0))
    h = jnp.zeros((1000, H), _f32)
    for r in range(R):
        h = h + jnp.dot(agg[r] * ndst[r], W[r], precision=_HIGH,
                        preferred_element_type=_f32)
    h = h + jnp.sum(b[...], axis=0)
    h = jnp.maximum(h, 0.0)
    nsrc = lax.rsqrt(jnp.maximum(doa[...] + dob[...], 1.0))
    f = h[None, :, :] * nsrc
    outA[...] = f[..., :Hh]
    outB[...] = f[..., Hh:]


def _k4(aggA, aggB, diA, diB, doA, doB, W, b, interpret=False):
    nb = N // 1000
    return pl.pallas_call(
        _k4_body,
        grid=(nb,),
        in_specs=[
            _agg_spec(),
            _agg_spec(),
            pl.BlockSpec((R, 1000, 1), lambda j: (0, j, 0)),
            pl.BlockSpec((R, 1000, 1), lambda j: (0, j, 0)),
            pl.BlockSpec((R, 1000, 1), lambda j: (0, j, 0)),
            pl.BlockSpec((R, 1000, 1), lambda j: (0, j, 0)),
            pl.BlockSpec((R, H, H), lambda j: (0, 0, 0)),
            pl.BlockSpec((R, 1, H), lambda j: (0, 0, 0)),
        ],
        out_specs=[pl.BlockSpec((R, 1000, Hh), lambda j: (0, j, 0))] * 2,
        out_shape=[jax.ShapeDtypeStruct((R, N, Hh), _f32)] * 2,
        interpret=interpret,
    )(aggA, aggB, diA, diB, doA, doB, W, b)


def _k6_body(aggA, aggB, dia, dib, W, b, gids, out, sums, counts):
    j = pl.program_id(0)
    nb = pl.num_programs(0)
    agg = jnp.concatenate([aggA[0], aggB[0]], axis=-1)
    ndst = lax.rsqrt(jnp.maximum(dia[...] + dib[...], 1.0))
    h = jnp.zeros((1000, H), _f32)
    for r in range(R):
        h = h + jnp.dot(agg[r] * ndst[r], W[r], precision=_HIGH,
                        preferred_element_type=_f32)
    h = h + jnp.sum(b[...], axis=0)
    giota = lax.broadcasted_iota(_i32, (1000, G), 1)
    P = (gids[...] == giota).astype(_f32)
    dn = (((0,), (0,)), ((), ()))
    psum = lax.dot_general(P, h, dn, precision=_HIGH,
                           preferred_element_type=_f32)
    pcnt = lax.dot_general(P, jnp.ones((1000, H), _f32), dn, precision=_HIGH,
                           preferred_element_type=_f32)

    @pl.when(j == 0)
    def _():
        sums[...] = psum
        counts[...] = pcnt

    @pl.when(j > 0)
    def _():
        sums[...] += psum
        counts[...] += pcnt

    @pl.when(j == nb - 1)
    def _():
        out[...] = sums[...] / jnp.maximum(counts[...], 1.0)


def _k6(aggA, aggB, diA, diB, W, b, gids, interpret=False):
    nb = N // 1000
    return pl.pallas_call(
        _k6_body,
        grid=(nb,),
        in_specs=[
            _agg_spec(),
            _agg_spec(),
            pl.BlockSpec((R, 1000, 1), lambda j: (0, j, 0)),
            pl.BlockSpec((R, 1000, 1), lambda j: (0, j, 0)),
            pl.BlockSpec((R, H, H), lambda j: (0, 0, 0)),
            pl.BlockSpec((R, 1, H), lambda j: (0, 0, 0)),
            pl.BlockSpec((1000, 1), lambda j: (j, 0)),
        ],
        out_specs=pl.BlockSpec((G, H), lambda j: (0, 0)),
        out_shape=jax.ShapeDtypeStruct((G, H), _f32),
        scratch_shapes=[pltpu.VMEM((G, H), _f32), pltpu.VMEM((G, H), _f32)],
        interpret=interpret,
    )(aggA, aggB, diA, diB, W, b, gids)


def kernel(x, edge_index, edge_type, graph_ids, W1, b1, W2, b2):
    src = edge_index[0].astype(_i32)
    dst = edge_index[1].astype(_i32)
    typ = edge_type.astype(_i32)

    pad = EPAD - E
    src_p = jnp.pad(src, (0, pad))
    dst_p = jnp.pad(dst, (0, pad), constant_values=10**8)
    typ_p = jnp.pad(typ, (0, pad))

    idx = _k0(src_p.reshape(-1, 128), dst_p.reshape(-1, 128),
              typ_p.reshape(-1, 128))
    fsrc_g, fsrc_d, fdst_d, lidx0, lidx1 = [a.reshape(-1) for a in idx]
    fsrc_g2 = fsrc_g.reshape(EPAD // CH, CH)
    lidx = jnp.stack([lidx0.reshape(EPAD // CH, CH),
                      lidx1.reshape(EPAD // CH, CH)], axis=0)

    degp = _k1(fsrc_d, fdst_d)
    doA = degp[0, :R * N].reshape(R * N, 1)
    doB = degp[1, :R * N].reshape(R * N, 1)
    diA = degp[0, DR:DR + R * N].reshape(R * N, 1)
    diB = degp[1, DR:DR + R * N].reshape(R * N, 1)
    diA3 = diA.reshape(R, N, 1)
    diB3 = diB.reshape(R, N, 1)
    doA3 = doA.reshape(R, N, 1)
    doB3 = doB.reshape(R, N, 1)

    F1a, F1b = _k2(x, doA, doB)
    o1a = _ksc_agg(F1a, fsrc_g2, lidx).reshape(NSC, R, SEG, Hh)
    o1b = _ksc_agg(F1b, fsrc_g2, lidx).reshape(NSC, R, SEG, Hh)

    F2a, F2b = _k4(o1a, o1b, diA3, diB3, doA3, doB3, W1, b1.reshape(R, 1, H))
    o2a = _ksc_agg(F2a.reshape(R * N, Hh), fsrc_g2, lidx).reshape(
        NSC, R, SEG, Hh)
    o2b = _ksc_agg(F2b.reshape(R * N, Hh), fsrc_g2, lidx).reshape(
        NSC, R, SEG, Hh)

    hg = _k6(o2a, o2b, diA3, diB3, W2, b2.reshape(R, 1, H),
             graph_ids.astype(_i32).reshape(N, 1))
    return hg
